# baseline (speedup 1.0000x reference)
import functools

import jax
import jax.numpy as jnp
from jax import lax
from jax.experimental import pallas as pl
from jax.experimental.pallas import tpu as pltpu
from jax.experimental.pallas import tpu_sc as plsc

VOCAB = 1000000
EMBED = 64
L = 200
B = 4096

NC = 2
NS = 16
NW = NC * NS

TOKENS = B * L
BLK = 128
NBB = B // BLK
NBLOCK = TOKENS // BLK
BLK_PER_W = NBLOCK // NW
TOK_PER_W = BLK_PER_W * BLK
NBUF = 5


def _emb_kernel(idx_hbm, tok_hbm, pos_hbm, out_hbm,
                idx_v, pos_v, bufs, tbufs, gsems, osems):
    wid = lax.axis_index("s") * NC + lax.axis_index("c")
    blk0 = wid * BLK_PER_W

    pltpu.sync_copy(idx_hbm.at[pl.ds(blk0 * BLK, TOK_PER_W)], idx_v)
    pltpu.sync_copy(pos_hbm, pos_v)

    def gather_start(j, b):
        pltpu.async_copy(
            tok_hbm.at[idx_v.at[pl.ds(j * BLK, BLK)]], bufs.at[b], gsems.at[b]
        )

    def gather_wait(b):
        pltpu.make_async_copy(
            tok_hbm.at[idx_v.at[pl.ds(0, BLK)]], bufs.at[b], gsems.at[b]
        ).wait()

    def out_start(j, b):
        blk = blk0 + j
        l = blk // NBB
        bb = blk - l * NBB
        pltpu.async_copy(
            tbufs.at[b],
            out_hbm.at[l, pl.ds(0, 8), bb, pl.ds(0, 8), pl.ds(0, BLK)],
            osems.at[b],
        )

    def out_wait(b):
        pltpu.make_async_copy(
            tbufs.at[b],
            out_hbm.at[0, pl.ds(0, 8), 0, pl.ds(0, 8), pl.ds(0, BLK)],
            osems.at[b],
        ).wait()

    iota = lax.iota(jnp.int32, 16)
    rot = [lax.rem(iota + k, 16) for k in range(16)]

    for b in range(NBUF):
        gather_start(b, b)

    def slot_body(j, b):
        blk = blk0 + j
        l = blk // NBB
        gather_wait(b)

        @pl.when(j >= NBUF)
        def _():
            out_wait(b)

        for fg in range(4):
            f0 = fg * 16
            pbase = jnp.full((16,), l * EMBED + f0, jnp.int32)
            pos_rots = [plsc.load_gather(pos_v, [pbase + rot[k]])
                        for k in range(16)]

            @plsc.parallel_loop(0, 8, unroll=1)
            def s_loop(s, _f0=f0, _pos=pos_rots):
                cv = iota + s * 16
                for k in range(16):
                    rv = rot[k] + _f0
                    val = plsc.load_gather(bufs.at[b], [cv, rv])
                    fr = lax.shift_right_logical(rv, 3)
                    fi = lax.bitwise_and(rv, 7)
                    plsc.store_scatter(
                        tbufs.at[b], [fr, fi, cv], val + _pos[k]
                    )

        out_start(j, b)

        @pl.when(j + NBUF < BLK_PER_W)
        def _():
            gather_start(j + NBUF, b)

    def round_body(g, carry):
        for b in range(NBUF):
            slot_body(g * NBUF + b, b)
        return carry

    lax.fori_loop(0, BLK_PER_W // NBUF, round_body, 0)

    for b in range(NBUF):
        out_wait(b)


@jax.jit
def _emb(idx_flat, token_table, pos_table):
    mesh = plsc.VectorSubcoreMesh(core_axis_name="c", subcore_axis_name="s")
    run = functools.partial(
        pl.kernel,
        mesh=mesh,
        compiler_params=pltpu.CompilerParams(
            use_tc_tiling_on_sc=False, needs_layout_passes=False,
            disable_bounds_checks=True, skip_device_barrier=True
        ),
        out_type=jax.ShapeDtypeStruct((L, 8, NBB, 8, BLK), jnp.float32),
        scratch_types=[
            pltpu.VMEM((TOK_PER_W,), jnp.int32),
            pltpu.VMEM((L * EMBED,), jnp.float32),
            pltpu.VMEM((NBUF, BLK, EMBED), jnp.float32),
            pltpu.VMEM((NBUF, 8, 8, BLK), jnp.float32),
            pltpu.SemaphoreType.DMA((NBUF,)),
            pltpu.SemaphoreType.DMA((NBUF,)),
        ],
    )(_emb_kernel)
    return run(idx_flat, token_table, pos_table)


def kernel(inputs, token_table, pos_table):
    idx_flat = jnp.swapaxes(inputs, 0, 1).reshape(-1).astype(jnp.int32)
    out5 = _emb(idx_flat, token_table, pos_table.reshape(-1))
    return out5.transpose(2, 4, 0, 1, 3).reshape(B, L, EMBED)


# speedup vs baseline: 1.4453x; 1.4453x over previous
import functools

import jax
import jax.numpy as jnp
from jax import lax
from jax.experimental import pallas as pl
from jax.experimental.pallas import tpu as pltpu
from jax.experimental.pallas import tpu_sc as plsc

VOCAB = 1000000
EMBED = 64
L = 200
B = 4096

NC = 2
NS = 16
NW = NC * NS

TOKENS = B * L
BLK = 128
NBB = B // BLK
NBLOCK = TOKENS // BLK
BLK_PER_W = NBLOCK // NW
TOK_PER_W = BLK_PER_W * BLK
NBUF = 4
SKEW = 136


def _emb_kernel(idx_hbm, tok_hbm, pos_hbm, out_hbm,
                idx_v, bufs, tbufs, pring, gsems, psems, osems):
    wid = lax.axis_index("s") * NC + lax.axis_index("c")
    blk0 = wid * BLK_PER_W

    pltpu.sync_copy(idx_hbm.at[pl.ds(blk0 * BLK, TOK_PER_W)], idx_v)

    def gather_start(j, b):
        l = (blk0 + j) // NBB
        pltpu.async_copy(
            tok_hbm.at[idx_v.at[pl.ds(j * BLK, BLK)]], bufs.at[b], gsems.at[b]
        )
        pltpu.async_copy(
            pos_hbm.at[pl.ds(l * EMBED, EMBED)], pring.at[b], psems.at[b]
        )

    def gather_wait(b):
        pltpu.make_async_copy(
            tok_hbm.at[idx_v.at[pl.ds(0, BLK)]], bufs.at[b], gsems.at[b]
        ).wait()
        pltpu.make_async_copy(
            pos_hbm.at[pl.ds(0, EMBED)], pring.at[b], psems.at[b]
        ).wait()

    def out_start(j, b):
        blk = blk0 + j
        l = blk // NBB
        bb = blk - l * NBB
        for fp in range(EMBED):
            pltpu.async_copy(
                tbufs.at[b, pl.ds(fp * SKEW, BLK)],
                out_hbm.at[l, fp >> 3, bb, fp & 7, pl.ds(0, BLK)],
                osems.at[b],
            )

    def out_wait(b):
        for fp in range(EMBED):
            pltpu.make_async_copy(
                tbufs.at[b, pl.ds(0, BLK)],
                out_hbm.at[0, 0, 0, 0, pl.ds(0, BLK)],
                osems.at[b],
            ).wait()

    iota = lax.iota(jnp.int32, 16)
    sk = [(iota + fg * 16) * SKEW for fg in range(4)]

    for b in range(NBUF):
        gather_start(b, b)

    def slot_body(j, b):
        gather_wait(b)

        @pl.when(j >= NBUF)
        def _():
            out_wait(b)

        pvec = [pring[b, pl.ds(fg * 16, 16)] for fg in range(4)]

        @plsc.parallel_loop(0, BLK, unroll=1)
        def bl_loop(bl):
            blv = jnp.full((16,), bl, jnp.int32)
            for fg in range(4):
                val = bufs[b, bl, pl.ds(fg * 16, 16)]
                plsc.store_scatter(
                    tbufs.at[b], [sk[fg] + blv], val + pvec[fg]
                )

        out_start(j, b)

        @pl.when(j + NBUF < BLK_PER_W)
        def _():
            gather_start(j + NBUF, b)

    def round_body(g, carry):
        for b in range(NBUF):
            slot_body(g * NBUF + b, b)
        return carry

    lax.fori_loop(0, BLK_PER_W // NBUF, round_body, 0)

    for b in range(NBUF):
        out_wait(b)


@jax.jit
def _emb(idx_flat, token_table, pos_flat):
    mesh = plsc.VectorSubcoreMesh(core_axis_name="c", subcore_axis_name="s")
    run = functools.partial(
        pl.kernel,
        mesh=mesh,
        compiler_params=pltpu.CompilerParams(
            use_tc_tiling_on_sc=False, needs_layout_passes=False,
            disable_bounds_checks=True, skip_device_barrier=True
        ),
        out_type=jax.ShapeDtypeStruct((L, 8, NBB, 8, BLK), jnp.float32),
        scratch_types=[
            pltpu.VMEM((TOK_PER_W,), jnp.int32),
            pltpu.VMEM((NBUF, BLK, EMBED), jnp.float32),
            pltpu.VMEM((NBUF, EMBED * SKEW), jnp.float32),
            pltpu.VMEM((NBUF, EMBED), jnp.float32),
            pltpu.SemaphoreType.DMA((NBUF,)),
            pltpu.SemaphoreType.DMA((NBUF,)),
            pltpu.SemaphoreType.DMA((NBUF,)),
        ],
    )(_emb_kernel)
    return run(idx_flat, token_table, pos_flat)


def kernel(inputs, token_table, pos_table):
    idx_flat = jnp.swapaxes(inputs, 0, 1).reshape(-1).astype(jnp.int32)
    out5 = _emb(idx_flat, token_table, pos_table.reshape(-1))
    return out5.transpose(2, 4, 0, 1, 3).reshape(B, L, EMBED)


# speedup vs baseline: 1.4487x; 1.0023x over previous
import functools

import jax
import jax.numpy as jnp
from jax import lax
from jax.experimental import pallas as pl
from jax.experimental.pallas import tpu as pltpu
from jax.experimental.pallas import tpu_sc as plsc

VOCAB = 1000000
EMBED = 64
L = 200
B = 4096

NC = 2
NS = 16
NW = NC * NS

TOKENS = B * L
BLK = 128
NBB = B // BLK
NBLOCK = TOKENS // BLK
BLK_PER_W = NBLOCK // NW
TOK_PER_W = BLK_PER_W * BLK
NBUF = 4
SKEW = 136


def _emb_kernel(idx_hbm, tok_hbm, pos_hbm, out_hbm,
                idx_v, bufs, tbufs, pring, gsems, psems, osems):
    wid = lax.axis_index("s") * NC + lax.axis_index("c")
    blk0 = wid * BLK_PER_W

    pltpu.sync_copy(idx_hbm.at[pl.ds(blk0 * BLK, TOK_PER_W)], idx_v)

    def gather_start(j, b):
        l = (blk0 + j) // NBB
        pltpu.async_copy(
            tok_hbm.at[idx_v.at[pl.ds(j * BLK, BLK)]], bufs.at[b], gsems.at[b]
        )
        pltpu.async_copy(
            pos_hbm.at[pl.ds(l * EMBED, EMBED)], pring.at[b], psems.at[b]
        )

    def gather_wait(b):
        pltpu.make_async_copy(
            tok_hbm.at[idx_v.at[pl.ds(0, BLK)]], bufs.at[b], gsems.at[b]
        ).wait()
        pltpu.make_async_copy(
            pos_hbm.at[pl.ds(0, EMBED)], pring.at[b], psems.at[b]
        ).wait()

    def out_start(j, b):
        blk = blk0 + j
        l = blk // NBB
        bb = blk - l * NBB
        for fp in range(EMBED):
            pltpu.async_copy(
                tbufs.at[b, pl.ds(fp * SKEW, BLK)],
                out_hbm.at[l, fp >> 3, bb, fp & 7, pl.ds(0, BLK)],
                osems.at[b],
            )

    def out_wait(b):
        for fp in range(EMBED):
            pltpu.make_async_copy(
                tbufs.at[b, pl.ds(0, BLK)],
                out_hbm.at[0, 0, 0, 0, pl.ds(0, BLK)],
                osems.at[b],
            ).wait()

    iota = lax.iota(jnp.int32, 16)
    sk = [(iota + fg * 16) * SKEW for fg in range(4)]

    for b in range(NBUF):
        gather_start(b, b)

    def slot_body(j, b):
        gather_wait(b)

        @pl.when(j >= NBUF)
        def _():
            out_wait(b)

        pvec = [pring[b, pl.ds(fg * 16, 16)] for fg in range(4)]

        @plsc.parallel_loop(0, BLK, unroll=1)
        def bl_loop(bl):
            blv = jnp.full((16,), bl, jnp.int32)
            for fg in range(4):
                val = bufs[b, bl, pl.ds(fg * 16, 16)]
                plsc.store_scatter(
                    tbufs.at[b], [sk[fg] + blv], val + pvec[fg]
                )

        out_start(j, b)

        @pl.when(j + NBUF < BLK_PER_W)
        def _():
            gather_start(j + NBUF, b)

    def round_body(g, carry):
        for b in range(NBUF):
            slot_body(g * NBUF + b, b)
        return carry

    lax.fori_loop(0, BLK_PER_W // NBUF, round_body, 0)

    for b in range(NBUF):
        out_wait(b)


@jax.jit
def _emb(idx_flat, token_table, pos_flat):
    mesh = plsc.VectorSubcoreMesh(core_axis_name="c", subcore_axis_name="s")
    run = functools.partial(
        pl.kernel,
        mesh=mesh,
        compiler_params=pltpu.CompilerParams(
            use_tc_tiling_on_sc=False, needs_layout_passes=False,
            disable_bounds_checks=True
        ),
        out_type=jax.ShapeDtypeStruct((L, 8, NBB, 8, BLK), jnp.float32),
        scratch_types=[
            pltpu.VMEM((TOK_PER_W,), jnp.int32),
            pltpu.VMEM((NBUF, BLK, EMBED), jnp.float32),
            pltpu.VMEM((NBUF, EMBED * SKEW), jnp.float32),
            pltpu.VMEM((NBUF, EMBED), jnp.float32),
            pltpu.SemaphoreType.DMA((NBUF,)),
            pltpu.SemaphoreType.DMA((NBUF,)),
            pltpu.SemaphoreType.DMA((NBUF,)),
        ],
    )(_emb_kernel)
    return run(idx_flat, token_table, pos_flat)


def kernel(inputs, token_table, pos_table):
    idx_flat = jnp.swapaxes(inputs, 0, 1).reshape(-1).astype(jnp.int32)
    out5 = _emb(idx_flat, token_table, pos_table.reshape(-1))
    return out5.transpose(2, 4, 0, 1, 3).reshape(B, L, EMBED)


# speedup vs baseline: 1.4736x; 1.0172x over previous
import functools

import jax
import jax.numpy as jnp
from jax import lax
from jax.experimental import pallas as pl
from jax.experimental.pallas import tpu as pltpu
from jax.experimental.pallas import tpu_sc as plsc

VOCAB = 1000000
EMBED = 64
L = 200
B = 4096

NC = 2
NS = 16
NW = NC * NS

TOKENS = B * L
BLK = 128
NBB = B // BLK
NBLOCK = TOKENS // BLK
BLK_PER_W = NBLOCK // NW
TOK_PER_W = BLK_PER_W * BLK
NBUF = 4
SKEW = 136


def _emb_kernel(idx_hbm, tok_hbm, pos_hbm, out_hbm,
                idx_v, bufs, tbufs, pring, gsems, psems, osems):
    wid = lax.axis_index("s") * NC + lax.axis_index("c")
    blk0 = wid * BLK_PER_W

    pltpu.sync_copy(idx_hbm.at[pl.ds(blk0 * BLK, TOK_PER_W)], idx_v)

    def gather_start(j, b):
        l = (blk0 + j) // NBB
        pltpu.async_copy(
            tok_hbm.at[idx_v.at[pl.ds(j * BLK, BLK)]], bufs.at[b], gsems.at[b]
        )
        pltpu.async_copy(
            pos_hbm.at[pl.ds(l * EMBED, EMBED)], pring.at[b], psems.at[b]
        )

    def gather_wait(b):
        pltpu.make_async_copy(
            tok_hbm.at[idx_v.at[pl.ds(0, BLK)]], bufs.at[b], gsems.at[b]
        ).wait()
        pltpu.make_async_copy(
            pos_hbm.at[pl.ds(0, EMBED)], pring.at[b], psems.at[b]
        ).wait()

    def out_start(j, b):
        blk = blk0 + j
        l = blk // NBB
        bb = blk - l * NBB
        for fp in range(EMBED):
            pltpu.async_copy(
                tbufs.at[b, pl.ds(fp * SKEW, BLK)],
                out_hbm.at[l, fp >> 3, bb, fp & 7, pl.ds(0, BLK)],
                osems.at[b],
            )

    def out_wait(b):
        for fp in range(EMBED):
            pltpu.make_async_copy(
                tbufs.at[b, pl.ds(0, BLK)],
                out_hbm.at[0, 0, 0, 0, pl.ds(0, BLK)],
                osems.at[b],
            ).wait()

    iota = lax.iota(jnp.int32, 16)
    sk = [(iota + fg * 16) * SKEW for fg in range(4)]

    for b in range(NBUF):
        gather_start(b, b)

    def slot_body(j, b):
        gather_wait(b)

        @pl.when(j >= NBUF)
        def _():
            out_wait(b)

        pvec = [pring[b, pl.ds(fg * 16, 16)] for fg in range(4)]

        @plsc.parallel_loop(0, BLK, unroll=2)
        def bl_loop(bl):
            blv = jnp.full((16,), bl, jnp.int32)
            for fg in range(4):
                val = bufs[b, bl, pl.ds(fg * 16, 16)]
                plsc.store_scatter(
                    tbufs.at[b], [sk[fg] + blv], val + pvec[fg]
                )

        out_start(j, b)

        @pl.when(j + NBUF < BLK_PER_W)
        def _():
            gather_start(j + NBUF, b)

    def round_body(g, carry):
        for b in range(NBUF):
            slot_body(g * NBUF + b, b)
        return carry

    lax.fori_loop(0, BLK_PER_W // NBUF, round_body, 0)

    for b in range(NBUF):
        out_wait(b)


@jax.jit
def _emb(idx_flat, token_table, pos_flat):
    mesh = plsc.VectorSubcoreMesh(core_axis_name="c", subcore_axis_name="s")
    run = functools.partial(
        pl.kernel,
        mesh=mesh,
        compiler_params=pltpu.CompilerParams(
            use_tc_tiling_on_sc=False, needs_layout_passes=False,
            disable_bounds_checks=True
        ),
        out_type=jax.ShapeDtypeStruct((L, 8, NBB, 8, BLK), jnp.float32),
        scratch_types=[
            pltpu.VMEM((TOK_PER_W,), jnp.int32),
            pltpu.VMEM((NBUF, BLK, EMBED), jnp.float32),
            pltpu.VMEM((NBUF, EMBED * SKEW), jnp.float32),
            pltpu.VMEM((NBUF, EMBED), jnp.float32),
            pltpu.SemaphoreType.DMA((NBUF,)),
            pltpu.SemaphoreType.DMA((NBUF,)),
            pltpu.SemaphoreType.DMA((NBUF,)),
        ],
    )(_emb_kernel)
    return run(idx_flat, token_table, pos_flat)


def kernel(inputs, token_table, pos_table):
    idx_flat = jnp.swapaxes(inputs, 0, 1).reshape(-1).astype(jnp.int32)
    out5 = _emb(idx_flat, token_table, pos_table.reshape(-1))
    return out5.transpose(2, 4, 0, 1, 3).reshape(B, L, EMBED)
